# baseline (speedup 1.0000x reference)
import functools

import jax
import jax.numpy as jnp
from jax import lax
from jax.experimental import pallas as pl
from jax.experimental.pallas import tpu as pltpu
from jax.experimental.pallas import tpu_sc as plsc

NC = 2
NS = 16
NW = NC * NS
CHUNK = 80
NBUF = 2


def _sc_gather(idx3d, table):
    nw, chunks, chunk = idx3d.shape
    d = table.shape[1]
    total = nw * chunks * chunk
    mesh = plsc.VectorSubcoreMesh(core_axis_name="c", subcore_axis_name="s")

    n_mid = (chunks - NBUF) // NBUF
    assert chunks - NBUF == n_mid * NBUF

    @functools.partial(
        pl.kernel,
        mesh=mesh,
        out_type=jax.ShapeDtypeStruct((total, d), jnp.float32),
        scratch_types=[
            pltpu.VMEM((chunks, chunk), jnp.int32),
            pltpu.VMEM((NBUF, chunk, d), jnp.float32),
        ]
        + [pltpu.SemaphoreType.DMA] * (2 * NBUF),
    )
    def k(idx_hbm, table_hbm, out_hbm, idx_v, rows_v, *sems):
        gsems = sems[:NBUF]
        ssems = sems[NBUF:]
        wid = lax.axis_index("s") * NC + lax.axis_index("c")
        crow = wid * chunks
        pltpu.sync_copy(idx_hbm.at[wid], idx_v)

        def gstart(c, b):
            pltpu.async_copy(table_hbm.at[idx_v.at[c]], rows_v.at[b], gsems[b])

        def gwait(b):
            pltpu.make_async_copy(
                table_hbm.at[idx_v.at[0]], rows_v.at[b], gsems[b]
            ).wait()

        def sstart(c, b):
            pltpu.async_copy(
                rows_v.at[b], out_hbm.at[pl.ds((crow + c) * chunk, chunk)], ssems[b]
            )

        def swait(b):
            pltpu.make_async_copy(
                rows_v.at[b], out_hbm.at[pl.ds(crow * chunk, chunk)], ssems[b]
            ).wait()

        def step(c, b, first, last):
            gwait(b)
            sstart(c, b)
            if not last:
                bn = (b + NBUF - 1) % NBUF
                if not first:
                    swait(bn)
                gstart(c + NBUF - 1, bn)

        for j in range(NBUF - 1):
            gstart(j, j)
        step(0, 0, first=True, last=False)

        def mid_group(g, carry):
            base = 1 + g * NBUF
            for j in range(NBUF):
                step(base + j, (1 + j) % NBUF, first=False, last=False)
            return carry

        lax.fori_loop(0, n_mid, mid_group, 0)

        for c in range(chunks - NBUF + 1, chunks):
            step(c, c % NBUF, first=False, last=True)

        for c in range(chunks - NBUF, chunks):
            swait(c % NBUF)

    return k(idx3d, table)


def kernel(x, encoding):
    b0, b1 = x.shape
    d = encoding.shape[1]
    idx3d = x.reshape(NW, (b0 * b1) // (NW * CHUNK), CHUNK)
    out = _sc_gather(idx3d, encoding)
    return out.reshape(b0, b1, d)


# speedup vs baseline: 1.0023x; 1.0023x over previous
import functools

import jax
import jax.numpy as jnp
from jax import lax
from jax.experimental import pallas as pl
from jax.experimental.pallas import tpu as pltpu
from jax.experimental.pallas import tpu_sc as plsc

NC = 2
NS = 16
NW = NC * NS
CHUNK = 40
NBUF = 4


def _sc_gather(idx3d, table):
    nw, chunks, chunk = idx3d.shape
    d = table.shape[1]
    total = nw * chunks * chunk
    mesh = plsc.VectorSubcoreMesh(core_axis_name="c", subcore_axis_name="s")

    n_mid = (chunks - NBUF) // NBUF
    assert chunks - NBUF == n_mid * NBUF

    @functools.partial(
        pl.kernel,
        mesh=mesh,
        out_type=jax.ShapeDtypeStruct((total, d), jnp.float32),
        scratch_types=[
            pltpu.VMEM((chunks, chunk), jnp.int32),
            pltpu.VMEM((NBUF, chunk, d), jnp.float32),
        ]
        + [pltpu.SemaphoreType.DMA] * (2 * NBUF),
    )
    def k(idx_hbm, table_hbm, out_hbm, idx_v, rows_v, *sems):
        gsems = sems[:NBUF]
        ssems = sems[NBUF:]
        wid = lax.axis_index("s") * NC + lax.axis_index("c")
        crow = wid * chunks
        pltpu.sync_copy(idx_hbm.at[wid], idx_v)

        def gstart(c, b):
            pltpu.async_copy(table_hbm.at[idx_v.at[c]], rows_v.at[b], gsems[b])

        def gwait(b):
            pltpu.make_async_copy(
                table_hbm.at[idx_v.at[0]], rows_v.at[b], gsems[b]
            ).wait()

        def sstart(c, b):
            pltpu.async_copy(
                rows_v.at[b], out_hbm.at[pl.ds((crow + c) * chunk, chunk)], ssems[b]
            )

        def swait(b):
            pltpu.make_async_copy(
                rows_v.at[b], out_hbm.at[pl.ds(crow * chunk, chunk)], ssems[b]
            ).wait()

        def step(c, b, first, last):
            gwait(b)
            sstart(c, b)
            if not last:
                bn = (b + NBUF - 1) % NBUF
                if not first:
                    swait(bn)
                gstart(c + NBUF - 1, bn)

        for j in range(NBUF - 1):
            gstart(j, j)
        step(0, 0, first=True, last=False)

        def mid_group(g, carry):
            base = 1 + g * NBUF
            for j in range(NBUF):
                step(base + j, (1 + j) % NBUF, first=False, last=False)
            return carry

        lax.fori_loop(0, n_mid, mid_group, 0)

        for c in range(chunks - NBUF + 1, chunks):
            step(c, c % NBUF, first=False, last=True)

        for c in range(chunks - NBUF, chunks):
            swait(c % NBUF)

    return k(idx3d, table)


def kernel(x, encoding):
    b0, b1 = x.shape
    d = encoding.shape[1]
    idx3d = x.reshape(NW, (b0 * b1) // (NW * CHUNK), CHUNK)
    out = _sc_gather(idx3d, encoding)
    return out.reshape(b0, b1, d)


# speedup vs baseline: 1.4677x; 1.4643x over previous
import functools

import jax
import jax.numpy as jnp
from jax import lax
from jax.experimental import pallas as pl
from jax.experimental.pallas import tpu as pltpu
from jax.experimental.pallas import tpu_sc as plsc

NC = 2
NS = 16
NW = NC * NS
NBUF = 2


def _sc_gather(idx3d, table):
    nw, chunks, chunk = idx3d.shape
    d = table.shape[1]
    mesh = plsc.VectorSubcoreMesh(core_axis_name="c", subcore_axis_name="s")

    n_mid = (chunks - NBUF) // NBUF
    assert chunks - NBUF == n_mid * NBUF

    @functools.partial(
        pl.kernel,
        mesh=mesh,
        out_type=jax.ShapeDtypeStruct((nw * chunks, chunk, d), jnp.float32),
        scratch_types=[
            pltpu.VMEM((chunks, chunk), jnp.int32),
            pltpu.VMEM((NBUF, chunk, d), jnp.float32),
        ]
        + [pltpu.SemaphoreType.DMA] * (2 * NBUF),
    )
    def k(idx_hbm, table_hbm, out_hbm, idx_v, rows_v, *sems):
        gsems = sems[:NBUF]
        ssems = sems[NBUF:]
        wid = lax.axis_index("s") * NC + lax.axis_index("c")
        crow = wid * chunks
        pltpu.sync_copy(idx_hbm.at[wid], idx_v)

        def gstart(c, b):
            pltpu.async_copy(table_hbm.at[idx_v.at[c]], rows_v.at[b], gsems[b])

        def gwait(b):
            pltpu.make_async_copy(
                table_hbm.at[idx_v.at[0]], rows_v.at[b], gsems[b]
            ).wait()

        def sstart(c, b):
            pltpu.async_copy(rows_v.at[b], out_hbm.at[crow + c], ssems[b])

        def swait(b):
            pltpu.make_async_copy(
                rows_v.at[b], out_hbm.at[crow], ssems[b]
            ).wait()

        def step(c, b, first, last):
            gwait(b)
            sstart(c, b)
            if not last:
                bn = (b + NBUF - 1) % NBUF
                if not first:
                    swait(bn)
                gstart(c + NBUF - 1, bn)

        for j in range(NBUF - 1):
            gstart(j, j)
        step(0, 0, first=True, last=False)

        def mid_group(g, carry):
            base = 1 + g * NBUF
            for j in range(NBUF):
                step(base + j, (1 + j) % NBUF, first=False, last=False)
            return carry

        lax.fori_loop(0, n_mid, mid_group, 0)

        for c in range(chunks - NBUF + 1, chunks):
            step(c, c % NBUF, first=False, last=True)

        for c in range(chunks - NBUF, chunks):
            swait(c % NBUF)

    return k(idx3d, table)


def kernel(x, encoding):
    b0, b1 = x.shape
    idx3d = x.reshape(NW, b0 // NW, b1)
    return _sc_gather(idx3d, encoding)


# speedup vs baseline: 1.5216x; 1.0368x over previous
import functools

import jax
import jax.numpy as jnp
from jax import lax
from jax.experimental import pallas as pl
from jax.experimental.pallas import tpu as pltpu
from jax.experimental.pallas import tpu_sc as plsc

NC = 2
NS = 16
NW = NC * NS
G = 40
NG = 5
NSL = 4
PERIOD = NG * G

_SLAB_NEEDS_G = (1, 2, 3, 4)
_G_NEEDS_S = (0, 1, 2, 3, 3)


def _sc_gather(idx3d, table, slabs_w, slab):
    nw, chunks, g = idx3d.shape
    d = table.shape[1]
    periods = slabs_w // NSL
    mesh = plsc.VectorSubcoreMesh(core_axis_name="c", subcore_axis_name="s")

    @functools.partial(
        pl.kernel,
        mesh=mesh,
        out_type=jax.ShapeDtypeStruct((nw * slabs_w, slab, d), jnp.float32),
        scratch_types=[
            pltpu.VMEM((chunks, g), jnp.int32),
            pltpu.VMEM((PERIOD * d // 128, 128), jnp.float32),
        ]
        + [pltpu.SemaphoreType.DMA] * (NG + NSL),
    )
    def k(idx_hbm, table_hbm, out_hbm, idx_v, ring_v, *sems):
        gsems = sems[:NG]
        ssems = sems[NG:]
        wid = lax.axis_index("s") * NC + lax.axis_index("c")
        cslab = wid * slabs_w
        pltpu.sync_copy(idx_hbm.at[wid], idx_v)

        r = d // 128

        def gdst(j):
            return ring_v.at[pl.ds(j * G * r, G * r)].reshape(G, d)

        def ssrc(j):
            return ring_v.at[pl.ds(j * slab * r, slab * r)].reshape(slab, d)

        def gstart(p, j):
            pltpu.async_copy(
                table_hbm.at[idx_v.at[NG * p + j]], gdst(j), gsems[j]
            )

        def gwait(j):
            pltpu.make_async_copy(
                table_hbm.at[idx_v.at[0]], gdst(j), gsems[j]
            ).wait()

        def sstart(p, j):
            pltpu.async_copy(
                ssrc(j), out_hbm.at[cslab + NSL * p + j], ssems[j]
            )

        def swait(j):
            pltpu.make_async_copy(
                ssrc(j), out_hbm.at[cslab], ssems[j]
            ).wait()

        def period(p, first):
            for j in range(NG):
                if not first:
                    for s in range(
                        _G_NEEDS_S[j - 1] + 1 if j else 0, _G_NEEDS_S[j] + 1
                    ):
                        swait(s)
                gstart(p, j)
            for j in range(NSL):
                for gg in range(
                    _SLAB_NEEDS_G[j - 1] + 1 if j else 0, _SLAB_NEEDS_G[j] + 1
                ):
                    gwait(gg)
                sstart(p, j)

        period(0, first=True)
        lax.fori_loop(1, periods, lambda p, c: (period(p, first=False), c)[1], 0)
        for j in range(NSL):
            swait(j)

    return k(idx3d, table)


def kernel(x, encoding):
    b0, b1 = x.shape
    slabs_w = b0 // NW
    idx3d = x.reshape(NW, (b0 * b1) // (NW * G), G)
    return _sc_gather(idx3d, encoding, slabs_w, b1)
